# Initial kernel scaffold; baseline (speedup 1.0000x reference)
import jax
import jax.numpy as jnp
from jax.experimental import pallas as pl


def kernel(x_layer, step, step_embedding):
    raise NotImplementedError("write your pallas kernel here")


# speedup vs baseline: 1.0118x; 1.0118x over previous
import jax
import jax.numpy as jnp
from jax.experimental import pallas as pl
from jax.experimental.pallas import tpu as pltpu

_NUM_CHANNELS = 2048
_SCALE = float(_NUM_CHANNELS) ** 0.5


def _body(step_ref, x_ref, emb_ref, o_ref):
    del step_ref
    o_ref[...] = x_ref[...] + emb_ref[0] * _SCALE


def kernel(x_layer, step, step_embedding):
    B, S, C = x_layer.shape
    N = B * S
    x2 = x_layer.reshape(N, C)
    emb = step_embedding.reshape(-1, 1, C)
    step_arr = jnp.atleast_1d(jnp.asarray(step, jnp.int32))

    rows = 1024
    grid = (N // rows,)

    out = pl.pallas_call(
        _body,
        grid_spec=pltpu.PrefetchScalarGridSpec(
            num_scalar_prefetch=1,
            grid=grid,
            in_specs=[
                pl.BlockSpec((rows, C), lambda i, s: (i, 0)),
                pl.BlockSpec((1, 1, C), lambda i, s: (s[0], 0, 0)),
            ],
            out_specs=pl.BlockSpec((rows, C), lambda i, s: (i, 0)),
        ),
        out_shape=jax.ShapeDtypeStruct((N, C), x_layer.dtype),
        compiler_params=pltpu.CompilerParams(
            dimension_semantics=("arbitrary",),
        ),
    )(step_arr, x2, emb)
    return out.reshape(B, S, C)
